# baseline (speedup 1.0000x reference)
import jax
import jax.numpy as jnp
from jax.experimental import pallas as pl

HIDDEN = 2048
NUM_EXPERTS = 8
TOP_K = 2
BLOCK = 1024


def _router_block(x_ref, w_ref, rw_ref, idx_ref, logits_ref):
    x = x_ref[...]
    w = w_ref[...]
    logits = jnp.dot(x, w, preferred_element_type=jnp.float32)
    logits_ref[...] = logits

    lane = jax.lax.broadcasted_iota(jnp.int32, logits.shape, 1)
    m1 = jnp.max(logits, axis=-1, keepdims=True)
    i1 = jnp.min(jnp.where(logits == m1, lane, NUM_EXPERTS), axis=-1,
                 keepdims=True)
    masked = jnp.where(lane == i1, -jnp.inf, logits)
    m2 = jnp.max(masked, axis=-1, keepdims=True)
    i2 = jnp.min(jnp.where(masked == m2, lane, NUM_EXPERTS), axis=-1,
                 keepdims=True)

    e2 = jnp.exp(m2 - m1)
    denom = 1.0 + e2
    rw_ref[...] = jnp.concatenate([1.0 / denom, e2 / denom], axis=-1)
    idx_ref[...] = jnp.concatenate([i1, i2], axis=-1)


def kernel(hidden_states, W_gate):
    B, S, H = hidden_states.shape
    T = B * S
    x = hidden_states.reshape(T, H)
    grid = (T // BLOCK,)

    rw, idx, logits = pl.pallas_call(
        _router_block,
        grid=grid,
        in_specs=[
            pl.BlockSpec((BLOCK, H), lambda i: (i, 0)),
            pl.BlockSpec((H, NUM_EXPERTS), lambda i: (0, 0)),
        ],
        out_specs=[
            pl.BlockSpec((BLOCK, TOP_K), lambda i: (i, 0)),
            pl.BlockSpec((BLOCK, TOP_K), lambda i: (i, 0)),
            pl.BlockSpec((BLOCK, NUM_EXPERTS), lambda i: (i, 0)),
        ],
        out_shape=[
            jax.ShapeDtypeStruct((T, TOP_K), jnp.float32),
            jax.ShapeDtypeStruct((T, TOP_K), jnp.int32),
            jax.ShapeDtypeStruct((T, NUM_EXPERTS), jnp.float32),
        ],
    )(x, W_gate)

    return (rw.reshape(B, S, TOP_K),
            idx.reshape(B, S, TOP_K),
            logits.reshape(B, S, NUM_EXPERTS))


# speedup vs baseline: 1.2853x; 1.2853x over previous
import jax
import jax.numpy as jnp
from jax.experimental import pallas as pl

HIDDEN = 2048
NUM_EXPERTS = 8
TOP_K = 2
BLOCK = 1024


def _gate_block(x_ref, w_ref, logits_ref):
    logits_ref[...] = jnp.dot(x_ref[...], w_ref[...],
                              preferred_element_type=jnp.float32)


def kernel(hidden_states, W_gate):
    B, S, H = hidden_states.shape
    T = B * S
    x = hidden_states.reshape(T, H)
    grid = (T // BLOCK,)

    logits = pl.pallas_call(
        _gate_block,
        grid=grid,
        in_specs=[
            pl.BlockSpec((BLOCK, H), lambda i: (i, 0)),
            pl.BlockSpec((H, NUM_EXPERTS), lambda i: (0, 0)),
        ],
        out_specs=pl.BlockSpec((BLOCK, NUM_EXPERTS), lambda i: (i, 0)),
        out_shape=jax.ShapeDtypeStruct((T, NUM_EXPERTS), jnp.float32),
    )(x, W_gate)

    logits = logits.reshape(B, S, NUM_EXPERTS)
    tw, ti = jax.lax.top_k(logits, TOP_K)
    rw = jax.nn.softmax(tw, axis=-1)
    return (rw, ti, logits)
